# baseline (speedup 1.0000x reference)
import functools

import jax
import jax.numpy as jnp
from jax import lax
from jax.experimental import pallas as pl
from jax.experimental.pallas import tpu as pltpu
from jax.experimental.pallas import tpu_sc as plsc

B = 4
N = 4096
L = 16
NC, NS = 2, 16
CHUNK = 1024
NG = CHUNK // L
XU = 16
NXB = N // XU


def _rbf16(v):
    c = v * jnp.float32(65537.0)
    return c - (c - v)


def _sc_body(p_hbm, out_hbm, xx, xy, xz, x2, yx, yy, yz, y2, macc):
    s = lax.axis_index("s")
    c = lax.axis_index("c")
    wid = s * NC + c
    dirv = wid % 2
    b = (wid // 2) % B
    chunk = wid // (2 * B)
    xsel = 1 - dirv

    xbase = ((xsel * B + b) * 3) * N
    ybase = ((dirv * B + b) * 3) * N + chunk * CHUNK
    pltpu.sync_copy(p_hbm.at[pl.ds(xbase, N)], xx)
    pltpu.sync_copy(p_hbm.at[pl.ds(xbase + N, N)], xy)
    pltpu.sync_copy(p_hbm.at[pl.ds(xbase + 2 * N, N)], xz)
    pltpu.sync_copy(p_hbm.at[pl.ds(ybase, CHUNK)], yx)
    pltpu.sync_copy(p_hbm.at[pl.ds(ybase + N, CHUNK)], yy)
    pltpu.sync_copy(p_hbm.at[pl.ds(ybase + 2 * N, CHUNK)], yz)

    def prx(g, carry):
        sl = pl.ds(g * L, L)
        ax, ay, az = xx[sl], xy[sl], xz[sl]
        x2[sl] = ax * ax + ay * ay + az * az
        xx[sl] = _rbf16(ax) * -2.0
        xy[sl] = _rbf16(ay) * -2.0
        xz[sl] = _rbf16(az) * -2.0
        return carry

    lax.fori_loop(0, N // L, prx, 0)

    inf16 = jnp.full((L,), jnp.float32(jnp.inf), jnp.float32)

    def pry(g, carry):
        sl = pl.ds(g * L, L)
        ax, ay, az = yx[sl], yy[sl], yz[sl]
        y2[sl] = ax * ax + ay * ay + az * az
        yx[sl] = _rbf16(ax)
        yy[sl] = _rbf16(ay)
        yz[sl] = _rbf16(az)
        macc[sl] = inf16
        return carry

    lax.fori_loop(0, NG, pry, 0)

    def xblk(xb, carry):
        xsl = pl.ds(xb * XU, XU)
        bx, by_, bz, b2 = xx[xsl], xy[xsl], xz[xsl], x2[xsl]
        sx = [bx[j] for j in range(XU)]
        sy = [by_[j] for j in range(XU)]
        sz = [bz[j] for j in range(XU)]
        s2 = [b2[j] for j in range(XU)]

        def gbody(g, gc):
            sl = pl.ds(g * L, L)
            vx, vy, vz, v2 = yx[sl], yy[sl], yz[sl], y2[sl]
            m = macc[sl]
            for j in range(XU):
                d = (v2 + s2[j]) + vx * sx[j] + vy * sy[j] + vz * sz[j]
                m = jnp.minimum(m, d)
            macc[sl] = m
            return gc

        lax.fori_loop(0, NG, gbody, 0, unroll=2)
        return carry

    lax.fori_loop(0, NXB, xblk, 0)

    obase = (dirv * B + b) * N + chunk * CHUNK
    pltpu.sync_copy(macc, out_hbm.at[pl.ds(obase, CHUNK)])


_sc_mins = pl.kernel(
    _sc_body,
    out_type=jax.ShapeDtypeStruct((2 * B * N,), jnp.float32),
    mesh=plsc.VectorSubcoreMesh(core_axis_name="c", subcore_axis_name="s"),
    scratch_types=[
        pltpu.VMEM((N,), jnp.float32),
        pltpu.VMEM((N,), jnp.float32),
        pltpu.VMEM((N,), jnp.float32),
        pltpu.VMEM((N,), jnp.float32),
        pltpu.VMEM((CHUNK,), jnp.float32),
        pltpu.VMEM((CHUNK,), jnp.float32),
        pltpu.VMEM((CHUNK,), jnp.float32),
        pltpu.VMEM((CHUNK,), jnp.float32),
        pltpu.VMEM((CHUNK,), jnp.float32),
    ],
)


def _fin_body(x_ref, o_ref):
    r = jnp.sqrt(jnp.maximum(x_ref[...], 0.0))
    o_ref[0, 0] = jnp.sum(r) / jnp.float32(2 * B * N)


_finish = pl.pallas_call(
    _fin_body,
    out_shape=jax.ShapeDtypeStruct((1, 1), jnp.float32),
    out_specs=pl.BlockSpec(memory_space=pltpu.SMEM),
)


def kernel(template, source):
    p = jnp.stack(
        [jnp.transpose(template, (0, 2, 1)), jnp.transpose(source, (0, 2, 1))]
    ).reshape(-1)
    mins = _sc_mins(p)
    loss = _finish(mins.reshape(2 * B, N))
    return loss[0, 0]


# speedup vs baseline: 3.0488x; 3.0488x over previous
import jax
import jax.numpy as jnp
from jax.experimental import pallas as pl
from jax.experimental.pallas import tpu as pltpu

B = 4
N = 4096
TN = 256
NT = N // TN
KP = 128


MS = 512


def _tc_body(t3_ref, s3_ref, tb_ref, sb_ref, row_ref, col_ref, m2_ref):
    b = pl.program_id(0)
    nt = pl.program_id(1)
    tblk = t3_ref[0]
    t2 = jnp.sum(tblk * tblk, axis=1)

    def mloop(mi, rowacc):
        msl = pl.ds(mi * MS, MS)
        sblk = s3_ref[0, :, msl]
        s2 = jnp.sum(sblk * sblk, axis=0)
        c = jnp.dot(tb_ref[0], sb_ref[0, :, msl],
                    preferred_element_type=jnp.float32)
        m2_ref[...] = t2[:, None] + (s2[None, :] - 2.0 * c)
        m2 = m2_ref[...]
        colstep = jnp.min(m2_ref[...], axis=0).reshape(1, MS)
        csl = (pl.ds(b, 1), pl.ds(mi * MS, MS))

        @pl.when(nt == 0)
        def _():
            col_ref[csl] = colstep

        @pl.when(nt != 0)
        def _():
            col_ref[csl] = jnp.minimum(col_ref[csl], colstep)

        return jnp.minimum(rowacc, jnp.min(m2, axis=1))

    inf = jnp.full((TN,), jnp.float32(jnp.inf), jnp.float32)
    rowacc = jax.lax.fori_loop(0, N // MS, mloop, inf)
    row_ref[pl.ds(b, 1), pl.ds(nt * TN, TN)] = rowacc.reshape(1, TN)


_tc_call = pl.pallas_call(
    _tc_body,
    grid=(B, NT),
    in_specs=[
        pl.BlockSpec((1, TN, 3), lambda b, nt: (b, nt, 0)),
        pl.BlockSpec((1, 3, N), lambda b, nt: (b, 0, 0)),
        pl.BlockSpec((1, TN, KP), lambda b, nt: (b, nt, 0)),
        pl.BlockSpec((1, KP, N), lambda b, nt: (b, 0, 0)),
    ],
    out_specs=[
        pl.BlockSpec((B, N), lambda b, nt: (0, 0)),
        pl.BlockSpec((B, N), lambda b, nt: (0, 0)),
    ],
    out_shape=[
        jax.ShapeDtypeStruct((B, N), jnp.float32),
        jax.ShapeDtypeStruct((B, N), jnp.float32),
    ],
    scratch_shapes=[pltpu.VMEM((TN, MS), jnp.float32)],
)


def _fin_body(row_ref, col_ref, o_ref):
    tot = jnp.sum(jnp.sqrt(jnp.maximum(row_ref[...], 0.0))) + jnp.sum(
        jnp.sqrt(jnp.maximum(col_ref[...], 0.0))
    )
    o_ref[0, 0] = tot / jnp.float32(2 * B * N)


_finish = pl.pallas_call(
    _fin_body,
    out_shape=jax.ShapeDtypeStruct((1, 1), jnp.float32),
    out_specs=pl.BlockSpec(memory_space=pltpu.SMEM),
)


def kernel(template, source):
    tb = jnp.pad(template.astype(jnp.bfloat16), ((0, 0), (0, 0), (0, KP - 3)))
    sb = jnp.transpose(
        jnp.pad(source.astype(jnp.bfloat16), ((0, 0), (0, 0), (0, KP - 3))),
        (0, 2, 1),
    )
    row, col = _tc_call(template, jnp.transpose(source, (0, 2, 1)), tb, sb)
    loss = _finish(row, col)
    return loss[0, 0]


# speedup vs baseline: 3.9604x; 1.2990x over previous
import jax
import jax.numpy as jnp
from jax.experimental import pallas as pl
from jax.experimental.pallas import tpu as pltpu

B = 4
N = 4096
TN = 256
NT = N // TN
KP = 128
MS = 512
NM = N // MS


def _tc_body(t3_ref, s3T_ref, tb_ref, sb_ref, row_ref, col_ref, c_ref):
    b = pl.program_id(0)
    nt = pl.program_id(1)
    tblk = t3_ref[0]
    t2 = jnp.sum(tblk * tblk, axis=1)

    def mloop(mi, rowacc):
        msl = pl.ds(mi * MS, MS)
        sblk = s3T_ref[0, :, msl]
        s2 = jnp.sum(sblk * sblk, axis=0)
        c_ref[...] = jnp.dot(
            tb_ref[0], sb_ref[0, :, msl], preferred_element_type=jnp.float32
        )

        rowpath = s2[None, :] + c_ref[...]
        rp = jnp.minimum(
            jnp.minimum(rowpath[:, 0:128], rowpath[:, 128:256]),
            jnp.minimum(rowpath[:, 256:384], rowpath[:, 384:512]),
        )
        rowacc = jnp.minimum(rowacc, rp)

        colpath = t2[:, None] + c_ref[...]
        parts = [colpath[8 * i : 8 * i + 8, :] for i in range(TN // 8)]
        while len(parts) > 1:
            parts = [
                jnp.minimum(parts[2 * i], parts[2 * i + 1])
                for i in range(len(parts) // 2)
            ]
        cp = parts[0]

        @pl.when(nt == 0)
        def _():
            col_ref[:, msl] = cp

        @pl.when(nt != 0)
        def _():
            col_ref[:, msl] = jnp.minimum(col_ref[:, msl], cp)

        return rowacc

    inf = jnp.full((TN, 128), jnp.float32(jnp.inf), jnp.float32)
    rowacc = jax.lax.fori_loop(0, NM, mloop, inf)
    rowmin = t2 + jnp.min(rowacc, axis=1)
    row_ref[pl.ds(b, 1), pl.ds(nt * TN, TN)] = rowmin.reshape(1, TN)


_tc_call = pl.pallas_call(
    _tc_body,
    grid=(B, NT),
    in_specs=[
        pl.BlockSpec((1, TN, 3), lambda b, nt: (b, nt, 0)),
        pl.BlockSpec((1, 3, N), lambda b, nt: (b, 0, 0)),
        pl.BlockSpec((1, TN, KP), lambda b, nt: (b, nt, 0)),
        pl.BlockSpec((1, KP, N), lambda b, nt: (b, 0, 0)),
    ],
    out_specs=[
        pl.BlockSpec((B, N), lambda b, nt: (0, 0)),
        pl.BlockSpec((8, N), lambda b, nt: (b, 0)),
    ],
    out_shape=[
        jax.ShapeDtypeStruct((B, N), jnp.float32),
        jax.ShapeDtypeStruct((8 * B, N), jnp.float32),
    ],
    scratch_shapes=[pltpu.VMEM((TN, MS), jnp.float32)],
)


def _fin_body(row_ref, col_ref, s3T_ref, o_ref):
    s2 = jnp.sum(s3T_ref[...] * s3T_ref[...], axis=1)
    colm = jnp.min(col_ref[...].reshape(B, 8, N), axis=1) + s2
    tot = jnp.sum(jnp.sqrt(jnp.maximum(row_ref[...], 0.0))) + jnp.sum(
        jnp.sqrt(jnp.maximum(colm, 0.0))
    )
    o_ref[0, 0] = tot / jnp.float32(2 * B * N)


_finish = pl.pallas_call(
    _fin_body,
    out_shape=jax.ShapeDtypeStruct((1, 1), jnp.float32),
    out_specs=pl.BlockSpec(memory_space=pltpu.SMEM),
)


def kernel(template, source):
    s3T = jnp.transpose(source, (0, 2, 1))
    tb = jnp.pad(
        template.astype(jnp.bfloat16) * jnp.bfloat16(-2.0),
        ((0, 0), (0, 0), (0, KP - 3)),
    )
    sb = jnp.transpose(
        jnp.pad(source.astype(jnp.bfloat16), ((0, 0), (0, 0), (0, KP - 3))),
        (0, 2, 1),
    )
    row, col = _tc_call(template, s3T, tb, sb)
    loss = _finish(row, col, s3T)
    return loss[0, 0]


# speedup vs baseline: 7.2916x; 1.8411x over previous
import jax
import jax.numpy as jnp
from jax.experimental import pallas as pl
from jax.experimental.pallas import tpu as pltpu

B = 4
N = 4096
TN = 256
NT = N // TN
KP = 128
MS = 512
NM = N // MS


def _tc_body(t3_ref, s3T_ref, tb_ref, sb_ref, row_ref, col_ref, ca_ref, cb_ref):
    b = pl.program_id(0)
    nt = pl.program_id(1)
    tblk = t3_ref[0]
    t2 = jnp.sum(tblk * tblk, axis=1)

    @pl.when(nt == 0)
    def _():
        col_ref[...] = jnp.full((8, N), jnp.float32(jnp.inf), jnp.float32)

    rowacc = jnp.full((TN, 128), jnp.float32(jnp.inf), jnp.float32)
    for mi in range(NM):
        cref = ca_ref if mi % 2 == 0 else cb_ref
        msl = pl.ds(mi * MS, MS)
        sblk = s3T_ref[0, :, msl]
        s2 = jnp.sum(sblk * sblk, axis=0)
        cref[...] = jnp.dot(
            tb_ref[0], sb_ref[0, :, msl], preferred_element_type=jnp.float32
        )

        rowpath = s2[None, :] + cref[...]
        rp = jnp.minimum(
            jnp.minimum(rowpath[:, 0:128], rowpath[:, 128:256]),
            jnp.minimum(rowpath[:, 256:384], rowpath[:, 384:512]),
        )
        rowacc = jnp.minimum(rowacc, rp)

        colpath = t2[:, None] + cref[...]
        parts = [colpath[8 * i : 8 * i + 8, :] for i in range(TN // 8)]
        while len(parts) > 1:
            parts = [
                jnp.minimum(parts[2 * i], parts[2 * i + 1])
                for i in range(len(parts) // 2)
            ]
        col_ref[:, msl] = jnp.minimum(col_ref[:, msl], parts[0])

    rowmin = t2 + jnp.min(rowacc, axis=1)
    row_ref[pl.ds(b, 1), pl.ds(nt * TN, TN)] = rowmin.reshape(1, TN)


_tc_call = pl.pallas_call(
    _tc_body,
    grid=(B, NT),
    in_specs=[
        pl.BlockSpec((1, TN, 3), lambda b, nt: (b, nt, 0)),
        pl.BlockSpec((1, 3, N), lambda b, nt: (b, 0, 0)),
        pl.BlockSpec((1, TN, KP), lambda b, nt: (b, nt, 0)),
        pl.BlockSpec((1, KP, N), lambda b, nt: (b, 0, 0)),
    ],
    out_specs=[
        pl.BlockSpec((B, N), lambda b, nt: (0, 0)),
        pl.BlockSpec((8, N), lambda b, nt: (b, 0)),
    ],
    out_shape=[
        jax.ShapeDtypeStruct((B, N), jnp.float32),
        jax.ShapeDtypeStruct((8 * B, N), jnp.float32),
    ],
    scratch_shapes=[
        pltpu.VMEM((TN, MS), jnp.float32),
        pltpu.VMEM((TN, MS), jnp.float32),
    ],
)


def _fin_body(row_ref, col_ref, s3T_ref, o_ref):
    s2 = jnp.sum(s3T_ref[...] * s3T_ref[...], axis=1)
    colm = jnp.min(col_ref[...].reshape(B, 8, N), axis=1) + s2
    tot = jnp.sum(jnp.sqrt(jnp.maximum(row_ref[...], 0.0))) + jnp.sum(
        jnp.sqrt(jnp.maximum(colm, 0.0))
    )
    o_ref[0, 0] = tot / jnp.float32(2 * B * N)


_finish = pl.pallas_call(
    _fin_body,
    out_shape=jax.ShapeDtypeStruct((1, 1), jnp.float32),
    out_specs=pl.BlockSpec(memory_space=pltpu.SMEM),
)


def kernel(template, source):
    s3T = jnp.transpose(source, (0, 2, 1))
    tb = jnp.pad(
        template.astype(jnp.bfloat16) * jnp.bfloat16(-2.0),
        ((0, 0), (0, 0), (0, KP - 3)),
    )
    sb = jnp.transpose(
        jnp.pad(source.astype(jnp.bfloat16), ((0, 0), (0, 0), (0, KP - 3))),
        (0, 2, 1),
    )
    row, col = _tc_call(template, s3T, tb, sb)
    loss = _finish(row, col, s3T)
    return loss[0, 0]
